# baseline (speedup 1.0000x reference)
import functools

import jax
import jax.numpy as jnp
from jax import lax
from jax.experimental import pallas as pl
from jax.experimental.pallas import tpu as pltpu
from jax.experimental.pallas import tpu_sc as plsc

_N = 10000
_E = 320000
_D = 128

_NC = 2
_NS = 16
_NW = _NC * _NS
_B = 128
_NBLK = 80
_NPH = 2
_BPP = _NBLK // _NPH
_EPT = _NBLK * _B
_EPAD = _NW * _EPT
_NPAD = 10240
_RPT = _NPAD // _NS


def _logmap0_body(x_ref, o_ref):
    xb = x_ref[...]
    col = lax.broadcasted_iota(jnp.int32, xb.shape, 1)
    y = jnp.where(col == 0, 0.0, xb)
    y_norm = jnp.maximum(
        jnp.sqrt(jnp.sum(y * y, axis=1, keepdims=True)), 1e-15)
    theta = jnp.maximum(xb[:, 0:1], 1.0 + 1e-7)
    arc = jnp.log(theta + jnp.sqrt(jnp.clip(theta * theta - 1.0, 1e-15, None)))
    o_ref[...] = y * (arc / y_norm)


def _expmap0_body(p_ref, o_ref):
    u = p_ref[0] + p_ref[1]
    col = lax.broadcasted_iota(jnp.int32, u.shape, 1)
    xr = jnp.where(col == 0, 0.0, u)
    x_norm = jnp.maximum(
        jnp.sqrt(jnp.sum(xr * xr, axis=1, keepdims=True)), 1e-15)
    theta = jnp.clip(x_norm, -15.0, 15.0)
    sinh_t = 0.5 * (jnp.exp(theta) - jnp.exp(-theta))
    rest = sinh_t * xr / x_norm
    first = jnp.sqrt(
        jnp.clip(1.0 + jnp.sum(rest * rest, axis=1, keepdims=True), 1e-7, None))
    o_ref[...] = jnp.where(col == 0, first, rest)


def _sc_body(xt_hbm, rows_hbm, cols_hbm, w_hbm, zeros_hbm, out_hbm,
             cols_v, rows_v, w_v, buf0, buf1, gsem0, gsem1, ssem0, ssem1, acc):
    cid = lax.axis_index("c")
    sid = lax.axis_index("s")
    wid = sid * _NC + cid
    bufs = (buf0, buf1)
    gsems = (gsem0, gsem1)
    ssems = (ssem0, ssem1)

    pltpu.sync_copy(zeros_hbm, acc.at[pl.ds(sid * _RPT, _RPT)])
    plsc.subcore_barrier()

    def phase_body(p, carry0):
        pltpu.sync_copy(cols_hbm.at[wid, p], cols_v)
        pltpu.sync_copy(rows_hbm.at[wid, p], rows_v)
        pltpu.sync_copy(w_hbm.at[wid, p], w_v)

        pltpu.async_copy(xt_hbm.at[cols_v.at[0]], buf0, gsem0)

        def pair_body(i, carry):
            for b in range(2):
                j = i * 2 + b
                buf = bufs[b]
                pltpu.make_async_copy(
                    xt_hbm.at[cols_v.at[j]], buf, gsems[b]).wait()

                @pl.when(j + 1 < _BPP)
                def _():
                    @pl.when(j >= 1)
                    def _():
                        pltpu.make_async_copy(
                            bufs[1 - b], acc.at[rows_v.at[j]],
                            ssems[1 - b]).wait()
                    pltpu.async_copy(
                        xt_hbm.at[cols_v.at[j + 1]], bufs[1 - b], gsems[1 - b])

                def group_body(g, c2):
                    wvec = w_v[j, pl.ds(g * 16, 16)]
                    for l in range(16):
                        w = wvec[l]
                        e = g * 16 + l
                        for d in range(_D // 16):
                            buf[e, pl.ds(d * 16, 16)] = (
                                buf[e, pl.ds(d * 16, 16)] * w)
                    return c2

                lax.fori_loop(0, _B // 16, group_body, 0)
                pltpu.async_copy(buf, acc.at[rows_v.at[j]], ssems[b], add=True)
            return carry

        lax.fori_loop(0, _BPP // 2, pair_body, 0)
        pltpu.make_async_copy(buf0, acc.at[rows_v.at[_BPP - 2]], ssem0).wait()
        pltpu.make_async_copy(buf1, acc.at[rows_v.at[_BPP - 1]], ssem1).wait()
        return carry0

    lax.fori_loop(0, _NPH, phase_body, 0)
    plsc.subcore_barrier()
    pltpu.sync_copy(acc.at[pl.ds(sid * _RPT, _RPT)],
                    out_hbm.at[cid, pl.ds(sid * _RPT, _RPT)])


_sc_agg = functools.partial(
    pl.kernel,
    _sc_body,
    out_type=jax.ShapeDtypeStruct((_NC, _NPAD, _D), jnp.float32),
    mesh=plsc.VectorSubcoreMesh(core_axis_name="c", subcore_axis_name="s"),
    scratch_types=[
        pltpu.VMEM((_BPP, _B), jnp.int32),
        pltpu.VMEM((_BPP, _B), jnp.int32),
        pltpu.VMEM((_BPP, _B), jnp.float32),
        pltpu.VMEM((_B, _D), jnp.float32),
        pltpu.VMEM((_B, _D), jnp.float32),
        pltpu.SemaphoreType.DMA,
        pltpu.SemaphoreType.DMA,
        pltpu.SemaphoreType.DMA,
        pltpu.SemaphoreType.DMA,
        pltpu.VMEM_SHARED((_NPAD, _D), jnp.float32),
    ],
)()


def kernel(x, edge_index, edge_weight):
    xt = pl.pallas_call(
        _logmap0_body,
        out_shape=jax.ShapeDtypeStruct((_N, _D), jnp.float32),
        grid=(5,),
        in_specs=[pl.BlockSpec((_N // 5, _D), lambda i: (i, 0))],
        out_specs=pl.BlockSpec((_N // 5, _D), lambda i: (i, 0)),
    )(x)

    pad = _EPAD - _E
    rows = jnp.concatenate(
        [edge_index[0], jnp.zeros((pad,), jnp.int32)]).reshape(
            _NW, _NPH, _BPP, _B)
    cols = jnp.concatenate(
        [edge_index[1], jnp.zeros((pad,), jnp.int32)]).reshape(
            _NW, _NPH, _BPP, _B)
    w = jnp.concatenate(
        [edge_weight, jnp.zeros((pad,), jnp.float32)]).reshape(
            _NW, _NPH, _BPP, _B)
    zeros = jnp.zeros((_RPT, _D), jnp.float32)

    partials = _sc_agg(xt, rows, cols, w, zeros)

    out = pl.pallas_call(
        _expmap0_body,
        out_shape=jax.ShapeDtypeStruct((_N, _D), jnp.float32),
        grid=(5,),
        in_specs=[pl.BlockSpec((_NC, _N // 5, _D), lambda i: (0, i, 0))],
        out_specs=pl.BlockSpec((_N // 5, _D), lambda i: (i, 0)),
    )(partials)
    return out


# speedup vs baseline: 2.9903x; 2.9903x over previous
import functools

import jax
import jax.numpy as jnp
from jax import lax
from jax.experimental import pallas as pl
from jax.experimental.pallas import tpu as pltpu
from jax.experimental.pallas import tpu_sc as plsc

_N = 10000
_E = 320000
_D = 128

_NC = 2
_NS = 16
_NW = _NC * _NS
_B = 128
_NBLK = 80
_NPH = 2
_BPP = _NBLK // _NPH
_EPT = _NBLK * _B
_EPAD = _NW * _EPT
_NPAD = 10240
_RPT = _NPAD // _NS


def _logmap0_body(x_ref, o_ref):
    xb = x_ref[...]
    col = lax.broadcasted_iota(jnp.int32, xb.shape, 1)
    y = jnp.where(col == 0, 0.0, xb)
    y_norm = jnp.maximum(
        jnp.sqrt(jnp.sum(y * y, axis=1, keepdims=True)), 1e-15)
    theta = jnp.maximum(xb[:, 0:1], 1.0 + 1e-7)
    arc = jnp.log(theta + jnp.sqrt(jnp.clip(theta * theta - 1.0, 1e-15, None)))
    o_ref[...] = y * (arc / y_norm)


def _expmap0_body(p_ref, o_ref):
    u = p_ref[0] + p_ref[1]
    col = lax.broadcasted_iota(jnp.int32, u.shape, 1)
    xr = jnp.where(col == 0, 0.0, u)
    x_norm = jnp.maximum(
        jnp.sqrt(jnp.sum(xr * xr, axis=1, keepdims=True)), 1e-15)
    theta = jnp.clip(x_norm, -15.0, 15.0)
    sinh_t = 0.5 * (jnp.exp(theta) - jnp.exp(-theta))
    rest = sinh_t * xr / x_norm
    first = jnp.sqrt(
        jnp.clip(1.0 + jnp.sum(rest * rest, axis=1, keepdims=True), 1e-7, None))
    o_ref[...] = jnp.where(col == 0, first, rest)


def _sc_body(xt_hbm, rows_hbm, cols_hbm, w_hbm, zeros_hbm, out_hbm,
             cols_v, rows_v, w_v, buf0, buf1, gsem0, gsem1, ssem0, ssem1, acc):
    cid = lax.axis_index("c")
    sid = lax.axis_index("s")
    wid = sid * _NC + cid
    bufs = (buf0, buf1)
    gsems = (gsem0, gsem1)
    ssems = (ssem0, ssem1)

    pltpu.sync_copy(zeros_hbm, acc.at[pl.ds(sid * _RPT, _RPT)])
    plsc.subcore_barrier()

    def phase_body(p, carry0):
        pltpu.sync_copy(cols_hbm.at[wid, p], cols_v)
        pltpu.sync_copy(rows_hbm.at[wid, p], rows_v)
        pltpu.sync_copy(w_hbm.at[wid, p], w_v)

        pltpu.async_copy(xt_hbm.at[cols_v.at[0]], buf0, gsem0)

        def pair_body(i, carry):
            for b in range(2):
                j = i * 2 + b
                buf = bufs[b]
                pltpu.make_async_copy(
                    xt_hbm.at[cols_v.at[j]], buf, gsems[b]).wait()

                @pl.when(j + 1 < _BPP)
                def _():
                    @pl.when(j >= 1)
                    def _():
                        pltpu.make_async_copy(
                            bufs[1 - b], acc.at[rows_v.at[j]],
                            ssems[1 - b]).wait()
                    pltpu.async_copy(
                        xt_hbm.at[cols_v.at[j + 1]], bufs[1 - b], gsems[1 - b])

                def group_body(g, c2):
                    wvec = w_v[j, pl.ds(g * 16, 16)]
                    for l in range(16):
                        w = wvec[l]
                        e = g * 16 + l
                        for d in range(_D // 16):
                            buf[e, pl.ds(d * 16, 16)] = (
                                buf[e, pl.ds(d * 16, 16)] * w)
                    return c2

                lax.fori_loop(0, _B // 16, group_body, 0)
                pltpu.async_copy(buf, acc.at[rows_v.at[j]], ssems[b], add=True)
            return carry

        lax.fori_loop(0, _BPP // 2, pair_body, 0)
        pltpu.make_async_copy(buf0, acc.at[rows_v.at[_BPP - 2]], ssem0).wait()
        pltpu.make_async_copy(buf1, acc.at[rows_v.at[_BPP - 1]], ssem1).wait()
        return carry0

    lax.fori_loop(0, _NPH, phase_body, 0)
    plsc.subcore_barrier()
    pltpu.sync_copy(acc.at[pl.ds(sid * _RPT, _RPT)],
                    out_hbm.at[cid, pl.ds(sid * _RPT, _RPT)])


_sc_agg = functools.partial(
    pl.kernel,
    _sc_body,
    out_type=jax.ShapeDtypeStruct((_NC, _NPAD, _D), jnp.float32),
    mesh=plsc.VectorSubcoreMesh(core_axis_name="c", subcore_axis_name="s"),
    scratch_types=[
        pltpu.VMEM((_BPP, _B), jnp.int32),
        pltpu.VMEM((_BPP, _B), jnp.int32),
        pltpu.VMEM((_BPP, _B), jnp.float32),
        pltpu.VMEM((_B, _D), jnp.float32),
        pltpu.VMEM((_B, _D), jnp.float32),
        pltpu.SemaphoreType.DMA,
        pltpu.SemaphoreType.DMA,
        pltpu.SemaphoreType.DMA,
        pltpu.SemaphoreType.DMA,
        pltpu.VMEM_SHARED((_NPAD, _D), jnp.float32),
    ],
)()


def kernel(x, edge_index, edge_weight):
    xt = pl.pallas_call(
        _logmap0_body,
        out_shape=jax.ShapeDtypeStruct((_N, _D), jnp.float32),
        grid=(5,),
        in_specs=[pl.BlockSpec((_N // 5, _D), lambda i: (i, 0))],
        out_specs=pl.BlockSpec((_N // 5, _D), lambda i: (i, 0)),
    )(x)

    pad = _EPAD - _E
    pad_idx = jnp.arange(pad, dtype=jnp.int32)
    rows = jnp.concatenate(
        [edge_index[0], pad_idx % _NPAD]).reshape(
            _NW, _NPH, _BPP, _B)
    cols = jnp.concatenate(
        [edge_index[1], pad_idx % _N]).reshape(
            _NW, _NPH, _BPP, _B)
    w = jnp.concatenate(
        [edge_weight, jnp.zeros((pad,), jnp.float32)]).reshape(
            _NW, _NPH, _BPP, _B)
    zeros = jnp.zeros((_RPT, _D), jnp.float32)

    partials = _sc_agg(xt, rows, cols, w, zeros)

    out = pl.pallas_call(
        _expmap0_body,
        out_shape=jax.ShapeDtypeStruct((_N, _D), jnp.float32),
        grid=(5,),
        in_specs=[pl.BlockSpec((_NC, _N // 5, _D), lambda i: (0, i, 0))],
        out_specs=pl.BlockSpec((_N // 5, _D), lambda i: (i, 0)),
    )(partials)
    return out
